# Initial kernel scaffold; baseline (speedup 1.0000x reference)
import jax
import jax.numpy as jnp
from jax.experimental import pallas as pl


def kernel(x, edge_index, edge_attr, W_e, b_e, W_attn, W_n, b_n):
    raise NotImplementedError("write your pallas kernel here")


# speedup vs baseline: 3.2625x; 3.2625x over previous
import functools

import jax
import jax.numpy as jnp
from jax import lax
from jax.experimental import pallas as pl
from jax.experimental.pallas import tpu as pltpu
from jax.experimental.pallas import tpu_sc as plsc

N_NODES = 10000
N_EDGES = 320000
D_NODE = 128
D_EDGE = 16
NW = 32
EPW = N_EDGES // NW
K = 48
NBF = EPW // K
TAIL = EPW - NBF * K
N_PAD = 10112
RPT = N_PAD // 16


def _lane_bcast(v, e):
    return v.at[jnp.full((16,), e, jnp.int32)].get(mode="promise_in_bounds")


def _rsqrt_nr(v):
    b = plsc.bitcast(v, jnp.int32)
    y = plsc.bitcast(jnp.full((16,), 0x5F3759DF, jnp.int32) - (b >> 1), jnp.float32)
    for _ in range(3):
        y = y * (1.5 - 0.5 * v * y * y)
    return y


def _node_pre_body(x_ref, w1_ref, w2_ref, a_ref, b_ref, rs_ref, ss_ref):
    xb = x_ref[...]
    a_ref[...] = jnp.dot(xb, w1_ref[...], preferred_element_type=jnp.float32)
    b_ref[...] = jnp.dot(xb, w2_ref[...], preferred_element_type=jnp.float32)
    rs_ref[...] = jnp.sum(xb, axis=1)
    ss_ref[...] = jnp.sum(xb * xb, axis=1)


def _node_pre(x, w1, w2):
    r = 1024
    grid = (pl.cdiv(N_NODES, r),)
    return pl.pallas_call(
        _node_pre_body,
        grid=grid,
        in_specs=[
            pl.BlockSpec((r, D_NODE), lambda i: (i, 0)),
            pl.BlockSpec((D_NODE, D_NODE), lambda i: (0, 0)),
            pl.BlockSpec((D_NODE, D_NODE), lambda i: (0, 0)),
        ],
        out_specs=[
            pl.BlockSpec((r, D_NODE), lambda i: (i, 0)),
            pl.BlockSpec((r, D_NODE), lambda i: (i, 0)),
            pl.BlockSpec((r,), lambda i: (i,)),
            pl.BlockSpec((r,), lambda i: (i,)),
        ],
        out_shape=[
            jax.ShapeDtypeStruct((N_NODES, D_NODE), jnp.float32),
            jax.ShapeDtypeStruct((N_NODES, D_NODE), jnp.float32),
            jax.ShapeDtypeStruct((N_NODES,), jnp.float32),
            jax.ShapeDtypeStruct((N_NODES,), jnp.float32),
        ],
    )(x, w1, w2)


def _edge_pre_body(ea_ref, w16_ref, c_ref, es_ref, ess_ref):
    eb = ea_ref[...]
    c_ref[...] = jnp.dot(eb, w16_ref[...], preferred_element_type=jnp.float32)
    es_ref[...] = jnp.sum(eb, axis=1)
    ess_ref[...] = jnp.sum(eb * eb, axis=1)


def _edge_pre(edge_attr, w16):
    r = 2048
    grid = (pl.cdiv(N_EDGES, r),)
    return pl.pallas_call(
        _edge_pre_body,
        grid=grid,
        in_specs=[
            pl.BlockSpec((r, D_EDGE), lambda i: (i, 0)),
            pl.BlockSpec((D_EDGE, D_NODE), lambda i: (0, 0)),
        ],
        out_specs=[
            pl.BlockSpec((r, D_NODE), lambda i: (i, 0)),
            pl.BlockSpec((r,), lambda i: (i,)),
            pl.BlockSpec((r,), lambda i: (i,)),
        ],
        out_shape=[
            jax.ShapeDtypeStruct((N_EDGES, D_NODE), jnp.float32),
            jax.ShapeDtypeStruct((N_EDGES,), jnp.float32),
            jax.ShapeDtypeStruct((N_EDGES,), jnp.float32),
        ],
    )(edge_attr, w16)


def _sc_body(a_t, b_t, rs_h, ss_h, c_t, es_h, ess_h, src_h, dst_h,
             sw_h, be_h, wa_h, pac_h, den_h,
             rsv, ssv, srcv, dstv, srcv16, dstv16, esv, essv, ts, td, cv,
             swv, bev, wav, denl, acc, sem1, sem2):
    cid = lax.axis_index("c")
    sid = lax.axis_index("s")
    wid = sid * 2 + cid

    pltpu.sync_copy(rs_h, rsv)
    pltpu.sync_copy(ss_h, ssv)
    pltpu.sync_copy(sw_h, swv)
    pltpu.sync_copy(be_h, bev)
    pltpu.sync_copy(wa_h, wav)
    swl = [swv[pl.ds(16 * j, 16)] for j in range(8)]
    bel = [bev[pl.ds(16 * j, 16)] for j in range(8)]
    wal = [wav[pl.ds(16 * j, 16)] for j in range(8)]
    lanes = lax.broadcasted_iota(jnp.int32, (16,), 0)
    zeros16 = jnp.zeros((16,), jnp.float32)

    def _zden(i, carry):
        denl[pl.ds(i * 16, 16)] = zeros16
        return carry
    lax.fori_loop(0, N_PAD // 16, _zden, 0)

    def _zcv(i, carry):
        for j in range(D_NODE // 16):
            cv[i, pl.ds(16 * j, 16)] = zeros16
        return carry
    lax.fori_loop(0, K, _zcv, 0)
    for kk in range(RPT // K):
        pltpu.sync_copy(cv, acc.at[pl.ds(sid * RPT + kk * K, K)])
    pltpu.sync_copy(cv.at[pl.ds(0, RPT - (RPT // K) * K)],
                    acc.at[pl.ds(sid * RPT + (RPT // K) * K,
                                 RPT - (RPT // K) * K)])
    plsc.subcore_barrier()

    def _do_group(g, s16, d16):
        rss = plsc.load_gather(rsv, [s16])
        sss = plsc.load_gather(ssv, [s16])
        rsd = plsc.load_gather(rsv, [d16])
        ssd = plsc.load_gather(ssv, [d16])
        mu = (rss + esv[pl.ds(g * 16, 16)] + rsd) * (1.0 / 272.0)
        q = (sss + essv[pl.ds(g * 16, 16)] + ssd) * (1.0 / 272.0)
        v = q - mu * mu + 1e-5
        inv = _rsqrt_nr(v)
        sg = v * inv
        def _edge(e, tv):
            r = g * 16 + e
            mu_e = _lane_bcast(mu, e)
            sg_e = _lane_bcast(sg, e)
            accv = jnp.zeros((16,), jnp.float32)
            for j in range(8):
                c16 = pl.ds(16 * j, 16)
                z = ts[r, c16] + td[r, c16] + cv[r, c16]
                u = z - mu_e * swl[j] + sg_e * bel[j]
                u = jnp.where(u >= 0.0, u, 0.01 * u)
                cv[r, c16] = u
                accv = accv + u * wal[j]
            t = jnp.sum(accv)
            return jnp.where(lanes == e, t, tv)
        tv = lax.fori_loop(0, 16, _edge, jnp.zeros((16,), jnp.float32))
        ea = jnp.exp(tv * inv)
        wvv = ea * inv
        plsc.addupdate_scatter(denl, [d16], ea)

        def _scale(e, c):
            r = g * 16 + e
            w_e = _lane_bcast(wvv, e)
            for j in range(8):
                c16 = pl.ds(16 * j, 16)
                cv[r, c16] = cv[r, c16] * w_e
            return c
        lax.fori_loop(0, 16, _scale, 0)

    base0 = wid * EPW

    def _block(bi, carry):
        base = base0 + bi * K
        pltpu.sync_copy(src_h.at[pl.ds(base, K)], srcv)
        pltpu.sync_copy(dst_h.at[pl.ds(base, K)], dstv)
        pltpu.sync_copy(es_h.at[pl.ds(base, K)], esv)
        pltpu.sync_copy(ess_h.at[pl.ds(base, K)], essv)
        pltpu.sync_copy(c_t.at[pl.ds(base, K)], cv)
        cp1 = pltpu.async_copy(a_t.at[srcv], ts, sem1)
        cp2 = pltpu.async_copy(b_t.at[dstv], td, sem2)
        cp1.wait()
        cp2.wait()

        def _group(g, gc):
            _do_group(g, srcv[pl.ds(g * 16, 16)], dstv[pl.ds(g * 16, 16)])
            return gc
        lax.fori_loop(0, K // 16, _group, 0)
        pltpu.sync_copy(cv, acc.at[dstv], add=True)
        return carry
    lax.fori_loop(0, NBF, _block, 0)

    tbase = base0 + NBF * K
    pltpu.sync_copy(src_h.at[pl.ds(tbase, TAIL)], srcv16)
    pltpu.sync_copy(dst_h.at[pl.ds(tbase, TAIL)], dstv16)
    pltpu.sync_copy(es_h.at[pl.ds(tbase, TAIL)], esv.at[pl.ds(0, TAIL)])
    pltpu.sync_copy(ess_h.at[pl.ds(tbase, TAIL)], essv.at[pl.ds(0, TAIL)])
    pltpu.sync_copy(c_t.at[pl.ds(tbase, TAIL)], cv.at[pl.ds(0, TAIL)])
    cp1 = pltpu.async_copy(a_t.at[srcv16], ts.at[pl.ds(0, TAIL)], sem1)
    cp2 = pltpu.async_copy(b_t.at[dstv16], td.at[pl.ds(0, TAIL)], sem2)
    cp1.wait()
    cp2.wait()
    _do_group(0, srcv16[...], dstv16[...])
    pltpu.sync_copy(cv.at[pl.ds(0, TAIL)], acc.at[dstv16], add=True)

    plsc.subcore_barrier()
    pltpu.sync_copy(acc.at[pl.ds(sid * RPT, RPT)],
                    pac_h.at[cid, pl.ds(sid * RPT, RPT)])
    pltpu.sync_copy(denl, den_h.at[wid])


def _sc_edge_pass(a_t, b_t, rs, ss, c_t, es, ess, src, dst, sw, be, wa):
    mesh = plsc.VectorSubcoreMesh(core_axis_name="c", subcore_axis_name="s")
    fn = functools.partial(
        pl.kernel,
        out_type=[
            jax.ShapeDtypeStruct((2, N_PAD, D_NODE), jnp.float32),
            jax.ShapeDtypeStruct((NW, N_PAD), jnp.float32),
        ],
        mesh=mesh,
        scratch_types=[
            pltpu.VMEM((N_NODES,), jnp.float32),
            pltpu.VMEM((N_NODES,), jnp.float32),
            pltpu.VMEM((K,), jnp.int32),
            pltpu.VMEM((K,), jnp.int32),
            pltpu.VMEM((16,), jnp.int32),
            pltpu.VMEM((16,), jnp.int32),
            pltpu.VMEM((K,), jnp.float32),
            pltpu.VMEM((K,), jnp.float32),
            pltpu.VMEM((K, D_NODE), jnp.float32),
            pltpu.VMEM((K, D_NODE), jnp.float32),
            pltpu.VMEM((K, D_NODE), jnp.float32),
            pltpu.VMEM((D_NODE,), jnp.float32),
            pltpu.VMEM((D_NODE,), jnp.float32),
            pltpu.VMEM((D_NODE,), jnp.float32),
            pltpu.VMEM((N_PAD,), jnp.float32),
            pltpu.VMEM_SHARED((N_PAD, D_NODE), jnp.float32),
            pltpu.SemaphoreType.DMA,
            pltpu.SemaphoreType.DMA,
        ],
        compiler_params=pltpu.CompilerParams(needs_layout_passes=False),
    )(_sc_body)
    return fn(a_t, b_t, rs, ss, c_t, es, ess, src, dst, sw, be, wa)


def _node_post_body(x_ref, pac_ref, den_ref, wn_ref, bn_ref, o_ref):
    p = pac_ref[...]
    s = p[0] + p[1]
    den = jnp.sum(den_ref[...], axis=0)
    af = s / jnp.maximum(den, 1e-20)[:, None]
    feat = jnp.concatenate([x_ref[...], af], axis=1)
    mu = jnp.mean(feat, axis=1, keepdims=True)
    var = jnp.mean(feat * feat, axis=1, keepdims=True) - mu * mu
    nf = (feat - mu) * lax.rsqrt(var + 1e-5)
    o = jnp.dot(nf, wn_ref[...], preferred_element_type=jnp.float32) + bn_ref[...]
    o_ref[...] = jnp.where(o >= 0.0, o, 0.01 * o)


def _node_post(x, pac, den, wn, bn):
    r = 1024
    grid = (pl.cdiv(N_NODES, r),)
    return pl.pallas_call(
        _node_post_body,
        grid=grid,
        in_specs=[
            pl.BlockSpec((r, D_NODE), lambda i: (i, 0)),
            pl.BlockSpec((2, r, D_NODE), lambda i: (0, i, 0)),
            pl.BlockSpec((NW, r), lambda i: (0, i)),
            pl.BlockSpec((2 * D_NODE, D_NODE), lambda i: (0, 0)),
            pl.BlockSpec((1, D_NODE), lambda i: (0, 0)),
        ],
        out_specs=pl.BlockSpec((r, D_NODE), lambda i: (i, 0)),
        out_shape=jax.ShapeDtypeStruct((N_NODES, D_NODE), jnp.float32),
    )(x, pac, den, wn, bn)


@jax.jit
def kernel(x, edge_index, edge_attr, W_e, b_e, W_attn, W_n, b_n):
    src = edge_index[0]
    dst = edge_index[1]
    w1 = W_e[:D_NODE]
    w16 = W_e[D_NODE:D_NODE + D_EDGE]
    w2 = W_e[D_NODE + D_EDGE:]
    sw = jnp.sum(W_e, axis=0)
    wa = W_attn[:, 0]

    a_t, b_t, rs, ss = _node_pre(x, w1, w2)
    c_t, es, ess = _edge_pre(edge_attr, w16)
    pac, den = _sc_edge_pass(a_t, b_t, rs, ss, c_t, es, ess, src, dst,
                             sw, b_e, wa)
    return _node_post(x, pac, den, W_n, b_n.reshape(1, D_NODE))


# speedup vs baseline: 3.8868x; 1.1914x over previous
import functools

import jax
import jax.numpy as jnp
from jax import lax
from jax.experimental import pallas as pl
from jax.experimental.pallas import tpu as pltpu
from jax.experimental.pallas import tpu_sc as plsc

N_NODES = 10000
N_EDGES = 320000
D_NODE = 128
D_EDGE = 16
NW = 32
EPW = N_EDGES // NW
K = 48
NBF = EPW // K
TAIL = EPW - NBF * K
N_PAD = 10112
RPT = N_PAD // 16


def _lane_bcast(v, e):
    return v.at[jnp.full((16,), e, jnp.int32)].get(mode="promise_in_bounds")


def _rsqrt_nr(v):
    b = plsc.bitcast(v, jnp.int32)
    y = plsc.bitcast(jnp.full((16,), 0x5F3759DF, jnp.int32) - (b >> 1), jnp.float32)
    for _ in range(3):
        y = y * (1.5 - 0.5 * v * y * y)
    return y


def _node_pre_body(x_ref, w1_ref, w2_ref, a_ref, b_ref, rs_ref, ss_ref):
    xb = x_ref[...]
    a_ref[...] = jnp.dot(xb, w1_ref[...], preferred_element_type=jnp.float32)
    b_ref[...] = jnp.dot(xb, w2_ref[...], preferred_element_type=jnp.float32)
    rs_ref[...] = jnp.sum(xb, axis=1)
    ss_ref[...] = jnp.sum(xb * xb, axis=1)


def _node_pre(x, w1, w2):
    r = 1024
    grid = (pl.cdiv(N_NODES, r),)
    return pl.pallas_call(
        _node_pre_body,
        grid=grid,
        in_specs=[
            pl.BlockSpec((r, D_NODE), lambda i: (i, 0)),
            pl.BlockSpec((D_NODE, D_NODE), lambda i: (0, 0)),
            pl.BlockSpec((D_NODE, D_NODE), lambda i: (0, 0)),
        ],
        out_specs=[
            pl.BlockSpec((r, D_NODE), lambda i: (i, 0)),
            pl.BlockSpec((r, D_NODE), lambda i: (i, 0)),
            pl.BlockSpec((r,), lambda i: (i,)),
            pl.BlockSpec((r,), lambda i: (i,)),
        ],
        out_shape=[
            jax.ShapeDtypeStruct((N_NODES, D_NODE), jnp.float32),
            jax.ShapeDtypeStruct((N_NODES, D_NODE), jnp.float32),
            jax.ShapeDtypeStruct((N_NODES,), jnp.float32),
            jax.ShapeDtypeStruct((N_NODES,), jnp.float32),
        ],
    )(x, w1, w2)


def _edge_pre_body(ea_ref, w16_ref, c_ref, es_ref, ess_ref):
    eb = ea_ref[...]
    c_ref[...] = jnp.dot(eb, w16_ref[...], preferred_element_type=jnp.float32)
    es_ref[...] = jnp.sum(eb, axis=1)
    ess_ref[...] = jnp.sum(eb * eb, axis=1)


def _edge_pre(edge_attr, w16):
    r = 2048
    grid = (pl.cdiv(N_EDGES, r),)
    return pl.pallas_call(
        _edge_pre_body,
        grid=grid,
        in_specs=[
            pl.BlockSpec((r, D_EDGE), lambda i: (i, 0)),
            pl.BlockSpec((D_EDGE, D_NODE), lambda i: (0, 0)),
        ],
        out_specs=[
            pl.BlockSpec((r, D_NODE), lambda i: (i, 0)),
            pl.BlockSpec((r,), lambda i: (i,)),
            pl.BlockSpec((r,), lambda i: (i,)),
        ],
        out_shape=[
            jax.ShapeDtypeStruct((N_EDGES, D_NODE), jnp.float32),
            jax.ShapeDtypeStruct((N_EDGES,), jnp.float32),
            jax.ShapeDtypeStruct((N_EDGES,), jnp.float32),
        ],
    )(edge_attr, w16)


def _sc_body(a_t, b_t, rs_h, ss_h, c_t, es_h, ess_h, src_h, dst_h,
             sw_h, be_h, wa_h, pac_h, den_h,
             rsv, ssv, srcv, dstv, srcv16, dstv16, esv, essv, ts, td, cv,
             swv, bev, wav, denl, acc, sem1, sem2):
    cid = lax.axis_index("c")
    sid = lax.axis_index("s")
    wid = sid * 2 + cid

    pltpu.sync_copy(rs_h, rsv)
    pltpu.sync_copy(ss_h, ssv)
    pltpu.sync_copy(sw_h, swv)
    pltpu.sync_copy(be_h, bev)
    pltpu.sync_copy(wa_h, wav)
    swl = [swv[pl.ds(16 * j, 16)] for j in range(8)]
    bel = [bev[pl.ds(16 * j, 16)] for j in range(8)]
    wal = [wav[pl.ds(16 * j, 16)] for j in range(8)]
    lanes = lax.broadcasted_iota(jnp.int32, (16,), 0)
    zeros16 = jnp.zeros((16,), jnp.float32)

    def _zden(i, carry):
        denl[pl.ds(i * 16, 16)] = zeros16
        return carry
    lax.fori_loop(0, N_PAD // 16, _zden, 0)

    def _zcv(i, carry):
        for j in range(D_NODE // 16):
            cv[i, pl.ds(16 * j, 16)] = zeros16
        return carry
    lax.fori_loop(0, K, _zcv, 0)
    for kk in range(RPT // K):
        pltpu.sync_copy(cv, acc.at[pl.ds(sid * RPT + kk * K, K)])
    pltpu.sync_copy(cv.at[pl.ds(0, RPT - (RPT // K) * K)],
                    acc.at[pl.ds(sid * RPT + (RPT // K) * K,
                                 RPT - (RPT // K) * K)])
    plsc.subcore_barrier()

    def _do_group(g, s16, d16):
        rss = plsc.load_gather(rsv, [s16])
        sss = plsc.load_gather(ssv, [s16])
        rsd = plsc.load_gather(rsv, [d16])
        ssd = plsc.load_gather(ssv, [d16])
        mu = (rss + esv[pl.ds(g * 16, 16)] + rsd) * (1.0 / 272.0)
        q = (sss + essv[pl.ds(g * 16, 16)] + ssd) * (1.0 / 272.0)
        v = q - mu * mu + 1e-5
        inv = _rsqrt_nr(v)
        sg = v * inv
        def _edge(e, tv):
            r = g * 16 + e
            mu_e = _lane_bcast(mu, e)
            sg_e = _lane_bcast(sg, e)
            accv = jnp.zeros((16,), jnp.float32)
            for j in range(8):
                c16 = pl.ds(16 * j, 16)
                z = ts[r, c16] + td[r, c16] + cv[r, c16]
                u = z - mu_e * swl[j] + sg_e * bel[j]
                u = jnp.where(u >= 0.0, u, 0.01 * u)
                cv[r, c16] = u
                accv = accv + u * wal[j]
            t = jnp.sum(accv)
            return jnp.where(lanes == e, t, tv)
        tv = lax.fori_loop(0, 16, _edge, jnp.zeros((16,), jnp.float32))
        ea = jnp.exp(tv * inv)
        wvv = ea * inv
        plsc.addupdate_scatter(denl, [d16], ea)

        def _scale(e, c):
            r = g * 16 + e
            w_e = _lane_bcast(wvv, e)
            for j in range(8):
                c16 = pl.ds(16 * j, 16)
                cv[r, c16] = cv[r, c16] * w_e
            return c
        lax.fori_loop(0, 16, _scale, 0)

    base0 = wid * EPW

    def _block(bi, carry):
        base = base0 + bi * K
        cps = [
            pltpu.async_copy(src_h.at[pl.ds(base, K)], srcv, sem1),
            pltpu.async_copy(dst_h.at[pl.ds(base, K)], dstv, sem1),
            pltpu.async_copy(es_h.at[pl.ds(base, K)], esv, sem1),
            pltpu.async_copy(ess_h.at[pl.ds(base, K)], essv, sem1),
            pltpu.async_copy(c_t.at[pl.ds(base, K)], cv, sem1),
        ]
        for cp in cps:
            cp.wait()
        cp1 = pltpu.async_copy(a_t.at[srcv], ts, sem1)
        cp2 = pltpu.async_copy(b_t.at[dstv], td, sem2)
        cp1.wait()
        cp2.wait()

        def _group(g, gc):
            _do_group(g, srcv[pl.ds(g * 16, 16)], dstv[pl.ds(g * 16, 16)])
            return gc
        lax.fori_loop(0, K // 16, _group, 0)
        pltpu.sync_copy(cv, acc.at[dstv], add=True)
        return carry
    lax.fori_loop(0, NBF, _block, 0)

    tbase = base0 + NBF * K
    cps = [
        pltpu.async_copy(src_h.at[pl.ds(tbase, TAIL)], srcv16, sem1),
        pltpu.async_copy(dst_h.at[pl.ds(tbase, TAIL)], dstv16, sem1),
        pltpu.async_copy(es_h.at[pl.ds(tbase, TAIL)], esv.at[pl.ds(0, TAIL)], sem1),
        pltpu.async_copy(ess_h.at[pl.ds(tbase, TAIL)], essv.at[pl.ds(0, TAIL)], sem1),
        pltpu.async_copy(c_t.at[pl.ds(tbase, TAIL)], cv.at[pl.ds(0, TAIL)], sem1),
    ]
    for cp in cps:
        cp.wait()
    cp1 = pltpu.async_copy(a_t.at[srcv16], ts.at[pl.ds(0, TAIL)], sem1)
    cp2 = pltpu.async_copy(b_t.at[dstv16], td.at[pl.ds(0, TAIL)], sem2)
    cp1.wait()
    cp2.wait()
    _do_group(0, srcv16[...], dstv16[...])
    pltpu.sync_copy(cv.at[pl.ds(0, TAIL)], acc.at[dstv16], add=True)

    plsc.subcore_barrier()
    pltpu.sync_copy(acc.at[pl.ds(sid * RPT, RPT)],
                    pac_h.at[cid, pl.ds(sid * RPT, RPT)])
    pltpu.sync_copy(denl, den_h.at[wid])


def _sc_edge_pass(a_t, b_t, rs, ss, c_t, es, ess, src, dst, sw, be, wa):
    mesh = plsc.VectorSubcoreMesh(core_axis_name="c", subcore_axis_name="s")
    fn = functools.partial(
        pl.kernel,
        out_type=[
            jax.ShapeDtypeStruct((2, N_PAD, D_NODE), jnp.float32),
            jax.ShapeDtypeStruct((NW, N_PAD), jnp.float32),
        ],
        mesh=mesh,
        scratch_types=[
            pltpu.VMEM((N_NODES,), jnp.float32),
            pltpu.VMEM((N_NODES,), jnp.float32),
            pltpu.VMEM((K,), jnp.int32),
            pltpu.VMEM((K,), jnp.int32),
            pltpu.VMEM((16,), jnp.int32),
            pltpu.VMEM((16,), jnp.int32),
            pltpu.VMEM((K,), jnp.float32),
            pltpu.VMEM((K,), jnp.float32),
            pltpu.VMEM((K, D_NODE), jnp.float32),
            pltpu.VMEM((K, D_NODE), jnp.float32),
            pltpu.VMEM((K, D_NODE), jnp.float32),
            pltpu.VMEM((D_NODE,), jnp.float32),
            pltpu.VMEM((D_NODE,), jnp.float32),
            pltpu.VMEM((D_NODE,), jnp.float32),
            pltpu.VMEM((N_PAD,), jnp.float32),
            pltpu.VMEM_SHARED((N_PAD, D_NODE), jnp.float32),
            pltpu.SemaphoreType.DMA,
            pltpu.SemaphoreType.DMA,
        ],
        compiler_params=pltpu.CompilerParams(needs_layout_passes=False),
    )(_sc_body)
    return fn(a_t, b_t, rs, ss, c_t, es, ess, src, dst, sw, be, wa)


def _node_post_body(x_ref, pac_ref, den_ref, wn_ref, bn_ref, o_ref):
    p = pac_ref[...]
    s = p[0] + p[1]
    den = jnp.sum(den_ref[...], axis=0)
    af = s / jnp.maximum(den, 1e-20)[:, None]
    feat = jnp.concatenate([x_ref[...], af], axis=1)
    mu = jnp.mean(feat, axis=1, keepdims=True)
    var = jnp.mean(feat * feat, axis=1, keepdims=True) - mu * mu
    nf = (feat - mu) * lax.rsqrt(var + 1e-5)
    o = jnp.dot(nf, wn_ref[...], preferred_element_type=jnp.float32) + bn_ref[...]
    o_ref[...] = jnp.where(o >= 0.0, o, 0.01 * o)


def _node_post(x, pac, den, wn, bn):
    r = 1024
    grid = (pl.cdiv(N_NODES, r),)
    return pl.pallas_call(
        _node_post_body,
        grid=grid,
        in_specs=[
            pl.BlockSpec((r, D_NODE), lambda i: (i, 0)),
            pl.BlockSpec((2, r, D_NODE), lambda i: (0, i, 0)),
            pl.BlockSpec((NW, r), lambda i: (0, i)),
            pl.BlockSpec((2 * D_NODE, D_NODE), lambda i: (0, 0)),
            pl.BlockSpec((1, D_NODE), lambda i: (0, 0)),
        ],
        out_specs=pl.BlockSpec((r, D_NODE), lambda i: (i, 0)),
        out_shape=jax.ShapeDtypeStruct((N_NODES, D_NODE), jnp.float32),
    )(x, pac, den, wn, bn)


@jax.jit
def kernel(x, edge_index, edge_attr, W_e, b_e, W_attn, W_n, b_n):
    src = edge_index[0]
    dst = edge_index[1]
    w1 = W_e[:D_NODE]
    w16 = W_e[D_NODE:D_NODE + D_EDGE]
    w2 = W_e[D_NODE + D_EDGE:]
    sw = jnp.sum(W_e, axis=0)
    wa = W_attn[:, 0]

    a_t, b_t, rs, ss = _node_pre(x, w1, w2)
    c_t, es, ess = _edge_pre(edge_attr, w16)
    pac, den = _sc_edge_pass(a_t, b_t, rs, ss, c_t, es, ess, src, dst,
                             sw, b_e, wa)
    return _node_post(x, pac, den, W_n, b_n.reshape(1, D_NODE))


# speedup vs baseline: 4.0560x; 1.0435x over previous
import functools

import jax
import jax.numpy as jnp
from jax import lax
from jax.experimental import pallas as pl
from jax.experimental.pallas import tpu as pltpu
from jax.experimental.pallas import tpu_sc as plsc

N_NODES = 10000
N_EDGES = 320000
D_NODE = 128
D_EDGE = 16
NW = 32
EPW = N_EDGES // NW
K = 48
NBF = EPW // K
TAIL = EPW - NBF * K
N_PAD = 10112
RPT = N_PAD // 16


def _lane_bcast(v, e):
    return v.at[jnp.full((16,), e, jnp.int32)].get(mode="promise_in_bounds")


def _rsqrt_nr(v):
    b = plsc.bitcast(v, jnp.int32)
    y = plsc.bitcast(jnp.full((16,), 0x5F3759DF, jnp.int32) - (b >> 1), jnp.float32)
    for _ in range(3):
        y = y * (1.5 - 0.5 * v * y * y)
    return y


def _node_pre_body(x_ref, w1_ref, w2_ref, a_ref, b_ref, rs_ref, ss_ref):
    xb = x_ref[...]
    a_ref[...] = jnp.dot(xb, w1_ref[...], preferred_element_type=jnp.float32)
    b_ref[...] = jnp.dot(xb, w2_ref[...], preferred_element_type=jnp.float32)
    rs_ref[...] = jnp.sum(xb, axis=1)
    ss_ref[...] = jnp.sum(xb * xb, axis=1)


def _node_pre(x, w1, w2):
    r = 1024
    grid = (pl.cdiv(N_NODES, r),)
    return pl.pallas_call(
        _node_pre_body,
        grid=grid,
        in_specs=[
            pl.BlockSpec((r, D_NODE), lambda i: (i, 0)),
            pl.BlockSpec((D_NODE, D_NODE), lambda i: (0, 0)),
            pl.BlockSpec((D_NODE, D_NODE), lambda i: (0, 0)),
        ],
        out_specs=[
            pl.BlockSpec((r, D_NODE), lambda i: (i, 0)),
            pl.BlockSpec((r, D_NODE), lambda i: (i, 0)),
            pl.BlockSpec((r,), lambda i: (i,)),
            pl.BlockSpec((r,), lambda i: (i,)),
        ],
        out_shape=[
            jax.ShapeDtypeStruct((N_NODES, D_NODE), jnp.float32),
            jax.ShapeDtypeStruct((N_NODES, D_NODE), jnp.float32),
            jax.ShapeDtypeStruct((N_NODES,), jnp.float32),
            jax.ShapeDtypeStruct((N_NODES,), jnp.float32),
        ],
    )(x, w1, w2)


def _edge_pre_body(ea_ref, w16_ref, c_ref, es_ref, ess_ref):
    eb = ea_ref[...]
    c_ref[...] = jnp.dot(eb, w16_ref[...], preferred_element_type=jnp.float32)
    es_ref[...] = jnp.sum(eb, axis=1)
    ess_ref[...] = jnp.sum(eb * eb, axis=1)


def _edge_pre(edge_attr, w16):
    r = 2048
    grid = (pl.cdiv(N_EDGES, r),)
    return pl.pallas_call(
        _edge_pre_body,
        grid=grid,
        in_specs=[
            pl.BlockSpec((r, D_EDGE), lambda i: (i, 0)),
            pl.BlockSpec((D_EDGE, D_NODE), lambda i: (0, 0)),
        ],
        out_specs=[
            pl.BlockSpec((r, D_NODE), lambda i: (i, 0)),
            pl.BlockSpec((r,), lambda i: (i,)),
            pl.BlockSpec((r,), lambda i: (i,)),
        ],
        out_shape=[
            jax.ShapeDtypeStruct((N_EDGES, D_NODE), jnp.float32),
            jax.ShapeDtypeStruct((N_EDGES,), jnp.float32),
            jax.ShapeDtypeStruct((N_EDGES,), jnp.float32),
        ],
    )(edge_attr, w16)


def _sc_body(a_t, b_t, rs_h, ss_h, c_t, es_h, ess_h, src_h, dst_h,
             sw_h, be_h, wa_h, pac_h, den_h,
             rsv, ssv, srcv, dstv, srcv16, dstv16, esv, essv, ts, td, cv,
             swv, bev, wav, denl, acc, sem1, sem2):
    cid = lax.axis_index("c")
    sid = lax.axis_index("s")
    wid = sid * 2 + cid

    pltpu.sync_copy(rs_h, rsv)
    pltpu.sync_copy(ss_h, ssv)
    pltpu.sync_copy(sw_h, swv)
    pltpu.sync_copy(be_h, bev)
    pltpu.sync_copy(wa_h, wav)
    swl = [swv[pl.ds(16 * j, 16)] for j in range(8)]
    bel = [bev[pl.ds(16 * j, 16)] for j in range(8)]
    wal = [wav[pl.ds(16 * j, 16)] for j in range(8)]
    lanes = lax.broadcasted_iota(jnp.int32, (16,), 0)
    zeros16 = jnp.zeros((16,), jnp.float32)

    def _zden(i, carry):
        denl[pl.ds(i * 16, 16)] = zeros16
        return carry
    lax.fori_loop(0, N_PAD // 16, _zden, 0)

    def _zcv(i, carry):
        for j in range(D_NODE // 16):
            cv[i, pl.ds(16 * j, 16)] = zeros16
        return carry
    lax.fori_loop(0, K, _zcv, 0)
    for kk in range(RPT // K):
        pltpu.sync_copy(cv, acc.at[pl.ds(sid * RPT + kk * K, K)])
    pltpu.sync_copy(cv.at[pl.ds(0, RPT - (RPT // K) * K)],
                    acc.at[pl.ds(sid * RPT + (RPT // K) * K,
                                 RPT - (RPT // K) * K)])
    plsc.subcore_barrier()

    def _group_stats(g, s16, d16):
        rss = plsc.load_gather(rsv, [s16])
        sss = plsc.load_gather(ssv, [s16])
        rsd = plsc.load_gather(rsv, [d16])
        ssd = plsc.load_gather(ssv, [d16])
        mu = (rss + esv[pl.ds(g * 16, 16)] + rsd) * (1.0 / 272.0)
        q = (sss + essv[pl.ds(g * 16, 16)] + ssd) * (1.0 / 272.0)
        v = q - mu * mu + 1e-5
        inv = _rsqrt_nr(v)
        esv[pl.ds(g * 16, 16)] = mu
        essv[pl.ds(g * 16, 16)] = v * inv

    def _do_group(g, d16):
        mu = esv[pl.ds(g * 16, 16)]
        sg = essv[pl.ds(g * 16, 16)]
        inv = _rsqrt_nr(sg * sg)
        def _edge(e, tv):
            r = g * 16 + e
            mu_e = _lane_bcast(mu, e)
            sg_e = _lane_bcast(sg, e)
            accv = jnp.zeros((16,), jnp.float32)
            for j in range(8):
                c16 = pl.ds(16 * j, 16)
                z = ts[r, c16] + td[r, c16] + cv[r, c16]
                u = z - mu_e * swl[j] + sg_e * bel[j]
                u = jnp.where(u >= 0.0, u, 0.01 * u)
                cv[r, c16] = u
                accv = accv + u * wal[j]
            t = jnp.sum(accv)
            return jnp.where(lanes == e, t, tv)
        tv = lax.fori_loop(0, 16, _edge, jnp.zeros((16,), jnp.float32))
        ea = jnp.exp(tv * inv)
        wvv = ea * inv
        plsc.addupdate_scatter(denl, [d16], ea)

        def _scale(e, c):
            r = g * 16 + e
            w_e = _lane_bcast(wvv, e)
            for j in range(8):
                c16 = pl.ds(16 * j, 16)
                cv[r, c16] = cv[r, c16] * w_e
            return c
        lax.fori_loop(0, 16, _scale, 0)

    base0 = wid * EPW

    def _block(bi, carry):
        base = base0 + bi * K
        cps = [
            pltpu.async_copy(src_h.at[pl.ds(base, K)], srcv, sem1),
            pltpu.async_copy(dst_h.at[pl.ds(base, K)], dstv, sem1),
            pltpu.async_copy(es_h.at[pl.ds(base, K)], esv, sem1),
            pltpu.async_copy(ess_h.at[pl.ds(base, K)], essv, sem1),
            pltpu.async_copy(c_t.at[pl.ds(base, K)], cv, sem1),
        ]
        cps[0].wait()
        cps[1].wait()
        cp1 = pltpu.async_copy(a_t.at[srcv], ts, sem2)
        cp2 = pltpu.async_copy(b_t.at[dstv], td, sem2)
        cps[2].wait()
        cps[3].wait()

        def _stats(g, gc):
            _group_stats(g, srcv[pl.ds(g * 16, 16)], dstv[pl.ds(g * 16, 16)])
            return gc
        lax.fori_loop(0, K // 16, _stats, 0)

        cps[4].wait()
        cp1.wait()
        cp2.wait()

        def _group(g, gc):
            _do_group(g, dstv[pl.ds(g * 16, 16)])
            return gc
        lax.fori_loop(0, K // 16, _group, 0)
        pltpu.sync_copy(cv, acc.at[dstv], add=True)
        return carry
    lax.fori_loop(0, NBF, _block, 0)

    tbase = base0 + NBF * K
    cps = [
        pltpu.async_copy(src_h.at[pl.ds(tbase, TAIL)], srcv16, sem1),
        pltpu.async_copy(dst_h.at[pl.ds(tbase, TAIL)], dstv16, sem1),
        pltpu.async_copy(es_h.at[pl.ds(tbase, TAIL)], esv.at[pl.ds(0, TAIL)], sem1),
        pltpu.async_copy(ess_h.at[pl.ds(tbase, TAIL)], essv.at[pl.ds(0, TAIL)], sem1),
        pltpu.async_copy(c_t.at[pl.ds(tbase, TAIL)], cv.at[pl.ds(0, TAIL)], sem1),
    ]
    cps[0].wait()
    cps[1].wait()
    cp1 = pltpu.async_copy(a_t.at[srcv16], ts.at[pl.ds(0, TAIL)], sem2)
    cp2 = pltpu.async_copy(b_t.at[dstv16], td.at[pl.ds(0, TAIL)], sem2)
    cps[2].wait()
    cps[3].wait()
    _group_stats(0, srcv16[...], dstv16[...])
    cps[4].wait()
    cp1.wait()
    cp2.wait()
    _do_group(0, dstv16[...])
    pltpu.sync_copy(cv.at[pl.ds(0, TAIL)], acc.at[dstv16], add=True)

    plsc.subcore_barrier()
    pltpu.sync_copy(acc.at[pl.ds(sid * RPT, RPT)],
                    pac_h.at[cid, pl.ds(sid * RPT, RPT)])
    pltpu.sync_copy(denl, den_h.at[wid])


def _sc_edge_pass(a_t, b_t, rs, ss, c_t, es, ess, src, dst, sw, be, wa):
    mesh = plsc.VectorSubcoreMesh(core_axis_name="c", subcore_axis_name="s")
    fn = functools.partial(
        pl.kernel,
        out_type=[
            jax.ShapeDtypeStruct((2, N_PAD, D_NODE), jnp.float32),
            jax.ShapeDtypeStruct((NW, N_PAD), jnp.float32),
        ],
        mesh=mesh,
        scratch_types=[
            pltpu.VMEM((N_NODES,), jnp.float32),
            pltpu.VMEM((N_NODES,), jnp.float32),
            pltpu.VMEM((K,), jnp.int32),
            pltpu.VMEM((K,), jnp.int32),
            pltpu.VMEM((16,), jnp.int32),
            pltpu.VMEM((16,), jnp.int32),
            pltpu.VMEM((K,), jnp.float32),
            pltpu.VMEM((K,), jnp.float32),
            pltpu.VMEM((K, D_NODE), jnp.float32),
            pltpu.VMEM((K, D_NODE), jnp.float32),
            pltpu.VMEM((K, D_NODE), jnp.float32),
            pltpu.VMEM((D_NODE,), jnp.float32),
            pltpu.VMEM((D_NODE,), jnp.float32),
            pltpu.VMEM((D_NODE,), jnp.float32),
            pltpu.VMEM((N_PAD,), jnp.float32),
            pltpu.VMEM_SHARED((N_PAD, D_NODE), jnp.float32),
            pltpu.SemaphoreType.DMA,
            pltpu.SemaphoreType.DMA,
        ],
        compiler_params=pltpu.CompilerParams(needs_layout_passes=False),
    )(_sc_body)
    return fn(a_t, b_t, rs, ss, c_t, es, ess, src, dst, sw, be, wa)


def _node_post_body(x_ref, pac_ref, den_ref, wn_ref, bn_ref, o_ref):
    p = pac_ref[...]
    s = p[0] + p[1]
    den = jnp.sum(den_ref[...], axis=0)
    af = s / jnp.maximum(den, 1e-20)[:, None]
    feat = jnp.concatenate([x_ref[...], af], axis=1)
    mu = jnp.mean(feat, axis=1, keepdims=True)
    var = jnp.mean(feat * feat, axis=1, keepdims=True) - mu * mu
    nf = (feat - mu) * lax.rsqrt(var + 1e-5)
    o = jnp.dot(nf, wn_ref[...], preferred_element_type=jnp.float32) + bn_ref[...]
    o_ref[...] = jnp.where(o >= 0.0, o, 0.01 * o)


def _node_post(x, pac, den, wn, bn):
    r = 1024
    grid = (pl.cdiv(N_NODES, r),)
    return pl.pallas_call(
        _node_post_body,
        grid=grid,
        in_specs=[
            pl.BlockSpec((r, D_NODE), lambda i: (i, 0)),
            pl.BlockSpec((2, r, D_NODE), lambda i: (0, i, 0)),
            pl.BlockSpec((NW, r), lambda i: (0, i)),
            pl.BlockSpec((2 * D_NODE, D_NODE), lambda i: (0, 0)),
            pl.BlockSpec((1, D_NODE), lambda i: (0, 0)),
        ],
        out_specs=pl.BlockSpec((r, D_NODE), lambda i: (i, 0)),
        out_shape=jax.ShapeDtypeStruct((N_NODES, D_NODE), jnp.float32),
    )(x, pac, den, wn, bn)


@jax.jit
def kernel(x, edge_index, edge_attr, W_e, b_e, W_attn, W_n, b_n):
    src = edge_index[0]
    dst = edge_index[1]
    w1 = W_e[:D_NODE]
    w16 = W_e[D_NODE:D_NODE + D_EDGE]
    w2 = W_e[D_NODE + D_EDGE:]
    sw = jnp.sum(W_e, axis=0)
    wa = W_attn[:, 0]

    a_t, b_t, rs, ss = _node_pre(x, w1, w2)
    c_t, es, ess = _edge_pre(edge_attr, w16)
    pac, den = _sc_edge_pass(a_t, b_t, rs, ss, c_t, es, ess, src, dst,
                             sw, b_e, wa)
    return _node_post(x, pac, den, W_n, b_n.reshape(1, D_NODE))
